# baseline (speedup 1.0000x reference)
import functools

import jax
import jax.numpy as jnp
from jax import lax
from jax.experimental import pallas as pl
from jax.experimental.pallas import tpu as pltpu
from jax.experimental.pallas import tpu_sc as plsc

N = 10000
D = 128
HD = 64
NC = 2
NS = 16
NW = NC * NS
LN = 128
CH = 80
CHS = CH * NC
EROWS = NW * CH
EPAD = EROWS * LN
NB = 4
GROUPS = CHS // NB
TRASH = N
ACC_ROWS = 10240
DEG_ROWS = 10016
BLK = 1000

_SC_PARAMS = pltpu.CompilerParams(
    needs_layout_passes=False, use_tc_tiling_on_sc=False)


def _sc_degree(dst_t, zdeg):
    mesh = plsc.VectorSubcoreMesh(core_axis_name="c", subcore_axis_name="s")

    @functools.partial(
        pl.kernel,
        out_type=jax.ShapeDtypeStruct((NW, N), jnp.float32),
        mesh=mesh,
        scratch_types=[
            pltpu.VMEM((CH, LN), jnp.int32),
            pltpu.VMEM((DEG_ROWS,), jnp.float32),
        ],
        compiler_params=_SC_PARAMS,
    )
    def k(dst_hbm, z_hbm, out_hbm, idx_v, acc_v):
        c = lax.axis_index("c")
        s = lax.axis_index("s")
        wid = s * NC + c
        pltpu.sync_copy(z_hbm, acc_v)
        pltpu.sync_copy(dst_hbm.at[pl.ds(wid * CH, CH)], idx_v)
        ones = jnp.full((16,), 1.0, jnp.float32)

        def body(j, carry):
            for u in range(LN // 16):
                idx = idx_v[j, pl.ds(u * 16, 16)]
                plsc.addupdate_scatter(acc_v, [idx], ones)
            return carry

        lax.fori_loop(0, CH, body, 0)
        pltpu.sync_copy(acc_v.at[pl.ds(0, N)], out_hbm.at[wid])

    return k(dst_t, zdeg)


def _sc_agg(g, src_t, dst_t, zrows):
    mesh = plsc.VectorSubcoreMesh(core_axis_name="c", subcore_axis_name="s")

    @functools.partial(
        pl.kernel,
        out_type=jax.ShapeDtypeStruct((NC, N, HD), jnp.bfloat16),
        mesh=mesh,
        scratch_types=[
            pltpu.VMEM((CHS, LN), jnp.int32),
            pltpu.VMEM((CHS, LN), jnp.int32),
            [pltpu.VMEM((LN, HD), jnp.bfloat16) for _ in range(NB)],
            pltpu.VMEM((125, HD), jnp.bfloat16),
            pltpu.VMEM_SHARED((ACC_ROWS, HD), jnp.bfloat16),
            [pltpu.SemaphoreType.DMA for _ in range(NB)],
            [pltpu.SemaphoreType.DMA for _ in range(NB)],
        ],
        compiler_params=_SC_PARAMS,
    )
    def k(g_hbm, src_hbm, dst_hbm, z_hbm, out_hbm,
          src_v, dst_v, rows, stage_v, acc_s, gs, ss):
        c = lax.axis_index("c")
        s = lax.axis_index("s")
        pltpu.sync_copy(
            z_hbm, acc_s.at[pl.ds(s * (ACC_ROWS // NS), ACC_ROWS // NS)])
        pltpu.sync_copy(src_hbm.at[pl.ds(s * CHS, CHS)], src_v)
        pltpu.sync_copy(dst_hbm.at[pl.ds(s * CHS, CHS)], dst_v)
        plsc.subcore_barrier()
        gc = g_hbm.at[c]

        for b in range(NB):
            pltpu.async_copy(gc.at[src_v.at[b]], rows[b], gs[b])

        def body(g, carry):
            j0 = g * NB
            for b in range(NB):
                j = j0 + b
                pltpu.make_async_copy(gc.at[src_v.at[j]], rows[b], gs[b]).wait()
                pltpu.async_copy(rows[b], acc_s.at[dst_v.at[j]], ss[b], add=True)

            @pl.when(g + 1 < GROUPS)
            def _():
                for b in range(NB):
                    j = j0 + b
                    pltpu.make_async_copy(
                        rows[b], acc_s.at[dst_v.at[j]], ss[b]).wait()
                    pltpu.async_copy(gc.at[src_v.at[j + NB]], rows[b], gs[b])
            return carry

        lax.fori_loop(0, GROUPS, body, 0)
        for b in range(NB):
            pltpu.make_async_copy(
                rows[b], acc_s.at[dst_v.at[CHS - NB + b]], ss[b]).wait()
        plsc.subcore_barrier()
        for i in range(5):
            r0 = s * (N // NS) + i * 125
            pltpu.sync_copy(acc_s.at[pl.ds(r0, 125)], stage_v)
            pltpu.sync_copy(stage_v, out_hbm.at[c, pl.ds(r0, 125)])

    return k(g, src_t, dst_t, zrows)


def _sparsemax3(s0, s1, s2):
    z1 = jnp.maximum(s0, jnp.maximum(s1, s2))
    z3 = jnp.minimum(s0, jnp.minimum(s1, s2))
    z2 = s0 + s1 + s2 - z1 - z3
    i2 = (1.0 + 2.0 * z2 > z1 + z2).astype(jnp.float32)
    i3 = (1.0 + 3.0 * z3 > z1 + z2 + z3).astype(jnp.float32)
    kf = 1.0 + i2 + i3
    tau = jnp.where(
        kf == 3.0, (z1 + z2 + z3 - 1.0) / 3.0,
        jnp.where(kf == 2.0, (z1 + z2 - 1.0) * 0.5, z1 - 1.0))
    w0 = jnp.maximum(s0 - tau, 0.0)
    w1 = jnp.maximum(s1 - tau, 0.0)
    w2 = jnp.maximum(s2 - tau, 0.0)
    return w0, w1, w2


def _proj_body(x_ref, w_ref, b_ref, degt_ref, g0_ref, dinv_ref):
    deg = jnp.sum(degt_ref[...], axis=1, keepdims=True) + 1.0
    dinv = lax.rsqrt(deg)
    x0 = jnp.dot(x_ref[...], w_ref[...], preferred_element_type=jnp.float32)
    x0 = jnp.maximum(x0 + b_ref[...], 0.0)
    g0 = (x0 * dinv).astype(jnp.bfloat16)
    g0_ref[0] = g0[:, :HD]
    g0_ref[1] = g0[:, HD:]
    dinv_ref[...] = dinv


def _tc_proj(x, w, b2, degt):
    return pl.pallas_call(
        _proj_body,
        grid=(N // BLK,),
        in_specs=[
            pl.BlockSpec((BLK, D), lambda i: (i, 0)),
            pl.BlockSpec((D, D), lambda i: (0, 0)),
            pl.BlockSpec((1, D), lambda i: (0, 0)),
            pl.BlockSpec((BLK, NW), lambda i: (i, 0)),
        ],
        out_specs=[
            pl.BlockSpec((NC, BLK, HD), lambda i: (0, i, 0)),
            pl.BlockSpec((BLK, 1), lambda i: (i, 0)),
        ],
        out_shape=[
            jax.ShapeDtypeStruct((NC, N, HD), jnp.bfloat16),
            jax.ShapeDtypeStruct((N, 1), jnp.float32),
        ],
    )(x, w, b2, degt)


def _conv_compute(acc_ref, g_ref, dinv, w0, w1, w2, att):
    xagg = jnp.concatenate(
        [acc_ref[0].astype(jnp.float32) + g_ref[0].astype(jnp.float32),
         acc_ref[1].astype(jnp.float32) + g_ref[1].astype(jnp.float32)],
        axis=1) * dinv
    h0 = jnp.dot(xagg, w0, preferred_element_type=jnp.float32)
    h1 = jnp.dot(xagg, w1, preferred_element_type=jnp.float32)
    h2 = jnp.dot(xagg, w2, preferred_element_type=jnp.float32)
    s0 = jnp.sum(h0 * att, axis=1, keepdims=True)
    s1 = jnp.sum(h1 * att, axis=1, keepdims=True)
    s2 = jnp.sum(h2 * att, axis=1, keepdims=True)
    b0, b1, b2 = _sparsemax3(s0, s1, s2)
    return jnp.maximum(b0 * h0 + b1 * h1 + b2 * h2, 0.0)


def _conv_body(acc_ref, g_ref, dinv_ref, w_ref, att_ref, out_ref):
    dinv = dinv_ref[...]
    x1 = _conv_compute(acc_ref, g_ref, dinv,
                       w_ref[0], w_ref[1], w_ref[2], att_ref[...])
    g1 = (x1 * dinv).astype(jnp.bfloat16)
    out_ref[0] = g1[:, :HD]
    out_ref[1] = g1[:, HD:]


def _tc_conv(acc, g, dinv, conv_w, att2):
    return pl.pallas_call(
        _conv_body,
        grid=(N // BLK,),
        in_specs=[
            pl.BlockSpec((NC, BLK, HD), lambda i: (0, i, 0)),
            pl.BlockSpec((NC, BLK, HD), lambda i: (0, i, 0)),
            pl.BlockSpec((BLK, 1), lambda i: (i, 0)),
            pl.BlockSpec((3, D, D), lambda i: (0, 0, 0)),
            pl.BlockSpec((1, D), lambda i: (0, 0)),
        ],
        out_specs=pl.BlockSpec((NC, BLK, HD), lambda i: (0, i, 0)),
        out_shape=jax.ShapeDtypeStruct((NC, N, HD), jnp.bfloat16),
    )(acc, g, dinv, conv_w, att2)


def _final_body(acc_ref, g_ref, dinv_ref, w_ref, att_ref,
                cw_ref, cb_ref, catt_ref, out_ref):
    dinv = dinv_ref[...]
    x2 = _conv_compute(acc_ref, g_ref, dinv,
                       w_ref[0], w_ref[1], w_ref[2], att_ref[...])
    catt = catt_ref[...]
    h0 = jnp.dot(x2, cw_ref[0], preferred_element_type=jnp.float32) + cb_ref[0:1, :]
    h1 = jnp.dot(x2, cw_ref[1], preferred_element_type=jnp.float32) + cb_ref[1:2, :]
    h2 = jnp.dot(x2, cw_ref[2], preferred_element_type=jnp.float32) + cb_ref[2:3, :]
    s0 = jnp.sum(h0 * catt, axis=1, keepdims=True)
    s1 = jnp.sum(h1 * catt, axis=1, keepdims=True)
    s2 = jnp.sum(h2 * catt, axis=1, keepdims=True)
    b0, b1, b2 = _sparsemax3(s0, s1, s2)
    out_ref[...] = b0 * h0 + b1 * h1 + b2 * h2


def _tc_final(acc, g, dinv, conv_w, att2, cwp, cbp, cattp):
    return pl.pallas_call(
        _final_body,
        grid=(N // BLK,),
        in_specs=[
            pl.BlockSpec((NC, BLK, HD), lambda i: (0, i, 0)),
            pl.BlockSpec((NC, BLK, HD), lambda i: (0, i, 0)),
            pl.BlockSpec((BLK, 1), lambda i: (i, 0)),
            pl.BlockSpec((3, D, D), lambda i: (0, 0, 0)),
            pl.BlockSpec((1, D), lambda i: (0, 0)),
            pl.BlockSpec((3, D, D), lambda i: (0, 0, 0)),
            pl.BlockSpec((3, D), lambda i: (0, 0)),
            pl.BlockSpec((1, D), lambda i: (0, 0)),
        ],
        out_specs=pl.BlockSpec((BLK, D), lambda i: (i, 0)),
        out_shape=jax.ShapeDtypeStruct((N, D), jnp.float32),
    )(acc, g, dinv, conv_w, att2, cwp, cbp, cattp)


def kernel(x, edge_index, W_lin, b_lin, conv1_w, conv1_att, conv2_w,
           conv2_att, cls_w, cls_b, cls_att):
    src = edge_index[0]
    dst = edge_index[1]
    e = src.shape[0]
    src_t = jnp.concatenate(
        [src, jnp.zeros((EPAD - e,), jnp.int32)]).reshape(EROWS, LN)
    dst_t = jnp.concatenate(
        [dst, jnp.full((EPAD - e,), TRASH, jnp.int32)]).reshape(EROWS, LN)
    zdeg = jnp.zeros((DEG_ROWS,), jnp.float32)
    zrows = jnp.zeros((ACC_ROWS // NS, HD), jnp.bfloat16)

    deg32 = _sc_degree(dst_t, zdeg)
    degt = deg32.T

    g0, dinv = _tc_proj(x, W_lin, b_lin.reshape(1, D), degt)
    acc1 = _sc_agg(g0, src_t, dst_t, zrows)
    g1 = _tc_conv(acc1, g0, dinv, conv1_w, conv1_att[:, 0].reshape(1, D))
    acc2 = _sc_agg(g1, src_t, dst_t, zrows)

    c = cls_w.shape[2]
    cwp = jnp.zeros((3, D, D), jnp.float32).at[:, :, :c].set(cls_w)
    cbp = jnp.zeros((3, D), jnp.float32).at[:, :c].set(cls_b)
    cattp = jnp.zeros((1, D), jnp.float32).at[0, :c].set(cls_att[:, 0])
    out = _tc_final(acc2, g1, dinv, conv2_w, conv2_att[:, 0].reshape(1, D),
                    cwp, cbp, cattp)
    return out[:, :c]


# speedup vs baseline: 1.5693x; 1.5693x over previous
import functools

import jax
import jax.numpy as jnp
from jax import lax
from jax.experimental import pallas as pl
from jax.experimental.pallas import tpu as pltpu
from jax.experimental.pallas import tpu_sc as plsc

N = 10000
D = 128
HD = 64
NC = 2
NS = 16
NW = NC * NS
LN = 128
CH = 80
CHS = CH * NC
EROWS = NW * CH
EPAD = EROWS * LN
NB = 4
GROUPS = CHS // NB
TRASH = N
ACC_ROWS = 10240
DEG_ROWS = 10240
BLK = 1000

_SC_PARAMS = pltpu.CompilerParams(
    needs_layout_passes=False, use_tc_tiling_on_sc=False)


def _sc_degree(dst_t, zdeg):
    mesh = plsc.VectorSubcoreMesh(core_axis_name="c", subcore_axis_name="s")

    @functools.partial(
        pl.kernel,
        out_type=jax.ShapeDtypeStruct((NW, N), jnp.float32),
        mesh=mesh,
        scratch_types=[
            pltpu.VMEM((CH, LN), jnp.int32),
            pltpu.VMEM((DEG_ROWS,), jnp.float32),
        ],
        compiler_params=_SC_PARAMS,
    )
    def k(dst_hbm, z_hbm, out_hbm, idx_v, acc_v):
        c = lax.axis_index("c")
        s = lax.axis_index("s")
        wid = s * NC + c
        pltpu.sync_copy(z_hbm, acc_v)
        pltpu.sync_copy(dst_hbm.at[pl.ds(wid * CH, CH)], idx_v)
        ones = jnp.full((16,), 1.0, jnp.float32)

        def body(j, carry):
            for u in range(LN // 16):
                idx = idx_v[j, pl.ds(u * 16, 16)]
                plsc.addupdate_scatter(acc_v, [idx], ones)
            return carry

        lax.fori_loop(0, CH, body, 0)
        pltpu.sync_copy(acc_v.at[pl.ds(0, N)], out_hbm.at[wid])

    return k(dst_t, zdeg)


def _sc_agg(g, src_t, dst_t, zrows):
    mesh = plsc.VectorSubcoreMesh(core_axis_name="c", subcore_axis_name="s")

    @functools.partial(
        pl.kernel,
        out_type=jax.ShapeDtypeStruct((NC, N, HD), jnp.bfloat16),
        mesh=mesh,
        scratch_types=[
            pltpu.VMEM((CHS, LN), jnp.int32),
            pltpu.VMEM((CHS, LN), jnp.int32),
            [pltpu.VMEM((LN, HD), jnp.bfloat16) for _ in range(NB)],
            pltpu.VMEM((125, HD), jnp.bfloat16),
            pltpu.VMEM_SHARED((ACC_ROWS, HD), jnp.bfloat16),
            pltpu.VMEM_SHARED((N, HD), jnp.bfloat16),
            [pltpu.SemaphoreType.DMA for _ in range(NB)],
            [pltpu.SemaphoreType.DMA for _ in range(NB)],
        ],
        compiler_params=_SC_PARAMS,
    )
    def k(g_hbm, src_hbm, dst_hbm, z_hbm, out_hbm,
          src_v, dst_v, rows, stage_v, acc_s, tbl_s, gs, ss):
        c = lax.axis_index("c")
        s = lax.axis_index("s")
        pltpu.sync_copy(
            z_hbm, acc_s.at[pl.ds(s * (ACC_ROWS // NS), ACC_ROWS // NS)])
        pltpu.sync_copy(g_hbm.at[c, pl.ds(s * (N // NS), N // NS)],
                        tbl_s.at[pl.ds(s * (N // NS), N // NS)])
        pltpu.sync_copy(src_hbm.at[pl.ds(s * CHS, CHS)], src_v)
        pltpu.sync_copy(dst_hbm.at[pl.ds(s * CHS, CHS)], dst_v)
        plsc.subcore_barrier()
        gc = tbl_s

        for b in range(NB):
            pltpu.async_copy(gc.at[src_v.at[b]], rows[b], gs[b])

        def body(g, carry):
            j0 = g * NB
            for b in range(NB):
                j = j0 + b
                pltpu.make_async_copy(gc.at[src_v.at[j]], rows[b], gs[b]).wait()
                pltpu.async_copy(rows[b], acc_s.at[dst_v.at[j]], ss[b], add=True)

            @pl.when(g + 1 < GROUPS)
            def _():
                for b in range(NB):
                    j = j0 + b
                    pltpu.make_async_copy(
                        rows[b], acc_s.at[dst_v.at[j]], ss[b]).wait()
                    pltpu.async_copy(gc.at[src_v.at[j + NB]], rows[b], gs[b])
            return carry

        lax.fori_loop(0, GROUPS, body, 0)
        for b in range(NB):
            pltpu.make_async_copy(
                rows[b], acc_s.at[dst_v.at[CHS - NB + b]], ss[b]).wait()
        plsc.subcore_barrier()
        for i in range(5):
            r0 = s * (N // NS) + i * 125
            pltpu.sync_copy(acc_s.at[pl.ds(r0, 125)], stage_v)
            pltpu.sync_copy(stage_v, out_hbm.at[c, pl.ds(r0, 125)])

    return k(g, src_t, dst_t, zrows)


def _sparsemax3(s0, s1, s2):
    z1 = jnp.maximum(s0, jnp.maximum(s1, s2))
    z3 = jnp.minimum(s0, jnp.minimum(s1, s2))
    z2 = s0 + s1 + s2 - z1 - z3
    i2 = (1.0 + 2.0 * z2 > z1 + z2).astype(jnp.float32)
    i3 = (1.0 + 3.0 * z3 > z1 + z2 + z3).astype(jnp.float32)
    kf = 1.0 + i2 + i3
    tau = jnp.where(
        kf == 3.0, (z1 + z2 + z3 - 1.0) / 3.0,
        jnp.where(kf == 2.0, (z1 + z2 - 1.0) * 0.5, z1 - 1.0))
    w0 = jnp.maximum(s0 - tau, 0.0)
    w1 = jnp.maximum(s1 - tau, 0.0)
    w2 = jnp.maximum(s2 - tau, 0.0)
    return w0, w1, w2


def _proj_body(x_ref, w_ref, b_ref, degt_ref, g0_ref, dinv_ref):
    deg = jnp.sum(degt_ref[...], axis=1, keepdims=True) + 1.0
    dinv = lax.rsqrt(deg)
    x0 = jnp.dot(x_ref[...], w_ref[...], preferred_element_type=jnp.float32)
    x0 = jnp.maximum(x0 + b_ref[...], 0.0)
    g0 = (x0 * dinv).astype(jnp.bfloat16)
    g0_ref[0] = g0[:, :HD]
    g0_ref[1] = g0[:, HD:]
    dinv_ref[...] = dinv


def _tc_proj(x, w, b2, degt):
    return pl.pallas_call(
        _proj_body,
        grid=(N // BLK,),
        in_specs=[
            pl.BlockSpec((BLK, D), lambda i: (i, 0)),
            pl.BlockSpec((D, D), lambda i: (0, 0)),
            pl.BlockSpec((1, D), lambda i: (0, 0)),
            pl.BlockSpec((BLK, NW), lambda i: (i, 0)),
        ],
        out_specs=[
            pl.BlockSpec((NC, BLK, HD), lambda i: (0, i, 0)),
            pl.BlockSpec((BLK, 1), lambda i: (i, 0)),
        ],
        out_shape=[
            jax.ShapeDtypeStruct((NC, N, HD), jnp.bfloat16),
            jax.ShapeDtypeStruct((N, 1), jnp.float32),
        ],
    )(x, w, b2, degt)


def _conv_compute(acc_ref, g_ref, dinv, w0, w1, w2, att):
    xagg = jnp.concatenate(
        [acc_ref[0].astype(jnp.float32) + g_ref[0].astype(jnp.float32),
         acc_ref[1].astype(jnp.float32) + g_ref[1].astype(jnp.float32)],
        axis=1) * dinv
    h0 = jnp.dot(xagg, w0, preferred_element_type=jnp.float32)
    h1 = jnp.dot(xagg, w1, preferred_element_type=jnp.float32)
    h2 = jnp.dot(xagg, w2, preferred_element_type=jnp.float32)
    s0 = jnp.sum(h0 * att, axis=1, keepdims=True)
    s1 = jnp.sum(h1 * att, axis=1, keepdims=True)
    s2 = jnp.sum(h2 * att, axis=1, keepdims=True)
    b0, b1, b2 = _sparsemax3(s0, s1, s2)
    return jnp.maximum(b0 * h0 + b1 * h1 + b2 * h2, 0.0)


def _conv_body(acc_ref, g_ref, dinv_ref, w_ref, att_ref, out_ref):
    dinv = dinv_ref[...]
    x1 = _conv_compute(acc_ref, g_ref, dinv,
                       w_ref[0], w_ref[1], w_ref[2], att_ref[...])
    g1 = (x1 * dinv).astype(jnp.bfloat16)
    out_ref[0] = g1[:, :HD]
    out_ref[1] = g1[:, HD:]


def _tc_conv(acc, g, dinv, conv_w, att2):
    return pl.pallas_call(
        _conv_body,
        grid=(N // BLK,),
        in_specs=[
            pl.BlockSpec((NC, BLK, HD), lambda i: (0, i, 0)),
            pl.BlockSpec((NC, BLK, HD), lambda i: (0, i, 0)),
            pl.BlockSpec((BLK, 1), lambda i: (i, 0)),
            pl.BlockSpec((3, D, D), lambda i: (0, 0, 0)),
            pl.BlockSpec((1, D), lambda i: (0, 0)),
        ],
        out_specs=pl.BlockSpec((NC, BLK, HD), lambda i: (0, i, 0)),
        out_shape=jax.ShapeDtypeStruct((NC, N, HD), jnp.bfloat16),
    )(acc, g, dinv, conv_w, att2)


def _final_body(acc_ref, g_ref, dinv_ref, w_ref, att_ref,
                cw_ref, cb_ref, catt_ref, out_ref):
    dinv = dinv_ref[...]
    x2 = _conv_compute(acc_ref, g_ref, dinv,
                       w_ref[0], w_ref[1], w_ref[2], att_ref[...])
    catt = catt_ref[...]
    h0 = jnp.dot(x2, cw_ref[0], preferred_element_type=jnp.float32) + cb_ref[0:1, :]
    h1 = jnp.dot(x2, cw_ref[1], preferred_element_type=jnp.float32) + cb_ref[1:2, :]
    h2 = jnp.dot(x2, cw_ref[2], preferred_element_type=jnp.float32) + cb_ref[2:3, :]
    s0 = jnp.sum(h0 * catt, axis=1, keepdims=True)
    s1 = jnp.sum(h1 * catt, axis=1, keepdims=True)
    s2 = jnp.sum(h2 * catt, axis=1, keepdims=True)
    b0, b1, b2 = _sparsemax3(s0, s1, s2)
    out_ref[...] = b0 * h0 + b1 * h1 + b2 * h2


def _tc_final(acc, g, dinv, conv_w, att2, cwp, cbp, cattp):
    return pl.pallas_call(
        _final_body,
        grid=(N // BLK,),
        in_specs=[
            pl.BlockSpec((NC, BLK, HD), lambda i: (0, i, 0)),
            pl.BlockSpec((NC, BLK, HD), lambda i: (0, i, 0)),
            pl.BlockSpec((BLK, 1), lambda i: (i, 0)),
            pl.BlockSpec((3, D, D), lambda i: (0, 0, 0)),
            pl.BlockSpec((1, D), lambda i: (0, 0)),
            pl.BlockSpec((3, D, D), lambda i: (0, 0, 0)),
            pl.BlockSpec((3, D), lambda i: (0, 0)),
            pl.BlockSpec((1, D), lambda i: (0, 0)),
        ],
        out_specs=pl.BlockSpec((BLK, D), lambda i: (i, 0)),
        out_shape=jax.ShapeDtypeStruct((N, D), jnp.float32),
    )(acc, g, dinv, conv_w, att2, cwp, cbp, cattp)


def kernel(x, edge_index, W_lin, b_lin, conv1_w, conv1_att, conv2_w,
           conv2_att, cls_w, cls_b, cls_att):
    src = edge_index[0]
    dst = edge_index[1]
    e = src.shape[0]
    pad = jnp.arange(EPAD - e, dtype=jnp.int32)
    src_t = jnp.concatenate([src, pad % N]).reshape(EROWS, LN)
    dst_t = jnp.concatenate(
        [dst, TRASH + pad % (ACC_ROWS - N)]).reshape(EROWS, LN)
    zdeg = jnp.zeros((DEG_ROWS,), jnp.float32)
    zrows = jnp.zeros((ACC_ROWS // NS, HD), jnp.bfloat16)

    deg32 = _sc_degree(dst_t, zdeg)
    degt = deg32.T

    g0, dinv = _tc_proj(x, W_lin, b_lin.reshape(1, D), degt)
    acc1 = _sc_agg(g0, src_t, dst_t, zrows)
    g1 = _tc_conv(acc1, g0, dinv, conv1_w, conv1_att[:, 0].reshape(1, D))
    acc2 = _sc_agg(g1, src_t, dst_t, zrows)

    c = cls_w.shape[2]
    cwp = jnp.zeros((3, D, D), jnp.float32).at[:, :, :c].set(cls_w)
    cbp = jnp.zeros((3, D), jnp.float32).at[:, :c].set(cls_b)
    cattp = jnp.zeros((1, D), jnp.float32).at[0, :c].set(cls_att[:, 0])
    out = _tc_final(acc2, g1, dinv, conv2_w, conv2_att[:, 0].reshape(1, D),
                    cwp, cbp, cattp)
    return out[:, :c]
